# Initial kernel scaffold; baseline (speedup 1.0000x reference)
import jax
import jax.numpy as jnp
from jax.experimental import pallas as pl


def kernel(x, perm):
    raise NotImplementedError("write your pallas kernel here")


# speedup vs baseline: 11.3400x; 11.3400x over previous
import functools

import jax
import jax.numpy as jnp
from jax import lax
from jax.experimental import pallas as pl
from jax.experimental.pallas import tpu as pltpu
from jax.experimental.pallas import tpu_sc as plsc

_NC = 2
_NS = 16
_NW = _NC * _NS
_LANES = 16
_WIN = 128


def kernel(x, perm):
    B, T, N, C = x.shape
    rows = B * T * N
    half = (T * N) // _NW
    x2 = x.reshape(rows, C)
    perm_flat = jnp.asarray(perm, jnp.int32).reshape(T * N)

    mesh = plsc.VectorSubcoreMesh(core_axis_name="c", subcore_axis_name="s")

    @functools.partial(
        pl.kernel,
        mesh=mesh,
        out_type=jax.ShapeDtypeStruct((rows, C), x.dtype),
        compiler_params=pltpu.CompilerParams(use_tc_tiling_on_sc=False),
        scratch_types=[
            pltpu.VMEM((half,), jnp.int32),
            pltpu.VMEM((half,), jnp.int32),
            pltpu.VMEM((half, C), jnp.float32),
            pltpu.SemaphoreType.DMA,
        ],
    )
    def scramble(x_hbm, perm_hbm, out_hbm, pidx_v, gidx_v, rows_v, sem):
        wid = lax.axis_index("s") * _NC + lax.axis_index("c")
        t = wid // 2
        h = wid % 2
        pltpu.sync_copy(perm_hbm.at[pl.ds(t * N + h * half, half)], pidx_v)

        @pl.loop(0, B)
        def _(b):
            off = (b * T + t) * N

            @pl.loop(0, half, step=_LANES)
            def _(i):
                gidx_v.at[pl.ds(i, _LANES)][...] = (
                    pidx_v.at[pl.ds(i, _LANES)][...] + off)

            copies = []
            for w in range(0, half, _WIN):
                copies.append(pltpu.async_copy(
                    x_hbm.at[gidx_v.at[pl.ds(w, _WIN)]],
                    rows_v.at[pl.ds(w, _WIN)], sem))
            for cp in copies:
                cp.wait()

            pltpu.sync_copy(
                rows_v,
                out_hbm.at[pl.ds((b * T + t) * N + h * half, half)])

    out2 = scramble(x2, perm_flat)
    return out2.reshape(B, T, N, C)
